# baseline (speedup 1.0000x reference)
import jax
import jax.numpy as jnp
from jax import lax
from jax.experimental import pallas as pl
from jax.experimental.pallas import tpu as pltpu
from jax.experimental.pallas import tpu_sc as plsc

SEQ = 200
D = 32
BATCH = 4096
SCALE = float(32.0 ** 0.5)

NC = 2
NS = 16
NW = NC * NS

BLK = 128
NBH = BATCH // BLK
N_TILES = SEQ * NBH
PER_W = N_TILES // NW
OUT_ROWS = SEQ * (D // 8) * NBH
RING = 8


def _lane_xor(x, k):
    idx = lax.iota(jnp.int32, 16) ^ k
    dn = lax.GatherDimensionNumbers(
        offset_dims=(), collapsed_slice_dims=(0,), start_index_map=(0,))
    return lax.gather(x, idx[:, None], dn, (1,),
                      mode=lax.GatherScatterMode.PROMISE_IN_BOUNDS)


def _transpose16(v):
    lane = lax.iota(jnp.int32, 16)
    for k in (8, 4, 2, 1):
        m = (lane & k) == 0
        nv = list(v)
        for i in range(16):
            if i & k:
                continue
            j = i | k
            x, y = v[i], v[j]
            mix = jnp.where(m, y, x)
            mixp = _lane_xor(mix, k)
            nv[i] = jnp.where(m, x, mixp)
            nv[j] = jnp.where(m, mixp, y)
        v = nv
    return v


def _body(idx_hbm, tok_hbm, pos_hbm, out_hbm, pos_v, *bufs):
    idx_v = bufs[0:RING]
    rows_v = bufs[RING:2 * RING]
    out_v = bufs[2 * RING:3 * RING]
    isem = bufs[3 * RING:4 * RING]
    gsem = bufs[4 * RING:5 * RING]
    osem = bufs[5 * RING:6 * RING]

    wid = lax.axis_index("s") * NC + lax.axis_index("c")
    base = wid * PER_W

    pltpu.sync_copy(pos_hbm, pos_v)

    def tile_sbh(g):
        t = base + g
        return t // NBH, t % NBH

    def idx_copy(g, r):
        s, bh = tile_sbh(g)
        return pltpu.make_async_copy(
            idx_hbm.at[s // 8, bh, s % 8], idx_v[r], isem[r])

    def gather_copy(r):
        return pltpu.make_async_copy(
            tok_hbm.at[idx_v[r]], rows_v[r], gsem[r])

    def out_copies(g, r):
        s, bh = tile_sbh(g)
        return [
            pltpu.make_async_copy(
                out_v[r].at[dh], out_hbm.at[(s * 4 + dh) * NBH + bh], osem[r])
            for dh in range(D // 8)
        ]

    def compute(g, r):
        s, _bh = tile_sbh(g)
        buf = rows_v[r]
        dst = out_v[r]

        def blk_body(blk, carry):
            bb = blk % 8
            db = blk // 8
            b0 = bb * 16
            pvec = pos_v[s, pl.ds(db * 16, 16)]
            v = [buf[b0 + l, pl.ds(db * 16, 16)] * SCALE + pvec
                 for l in range(16)]
            t = _transpose16(v)
            for dd in range(16):
                d = db * 16 + dd
                dst[d // 8, pl.ds((d % 8) * BLK + b0, 16)] = t[dd]
            return carry

        lax.fori_loop(0, 16, blk_body, 0)

    for c in range(RING):
        idx_copy(c, c).start()
    for c in range(RING - 1):
        idx_copy(c, c).wait()
        gather_copy(c).start()

    def loop_body(go, carry):
        for rr in range(RING):
            g = go * RING + rr
            gather_copy(rr).wait()

            @pl.when(g + RING < PER_W)
            def _():
                idx_copy(g + RING, rr).start()

            @pl.when(g + RING - 1 < PER_W)
            def _():
                idx_copy(g + RING - 1, (rr + RING - 1) % RING).wait()
                gather_copy((rr + RING - 1) % RING).start()

            @pl.when(g >= RING)
            def _():
                for cp in out_copies(g - RING, rr):
                    cp.wait()

            compute(g, rr)
            for cp in out_copies(g, rr):
                cp.start()
        return carry

    lax.fori_loop(0, PER_W // RING, loop_body, 0)

    for rr in range(RING):
        for cp in out_copies(PER_W - RING + rr, rr):
            cp.wait()


@jax.jit
def _embed(idx4, token_table, pos_table):
    mesh = plsc.VectorSubcoreMesh(core_axis_name="c", subcore_axis_name="s")
    return pl.kernel(
        _body,
        out_type=jax.ShapeDtypeStruct((OUT_ROWS, 8 * BLK), jnp.float32),
        mesh=mesh,
        compiler_params=pltpu.CompilerParams(use_tc_tiling_on_sc=False),
        scratch_types=(
            [pltpu.VMEM((SEQ, D), jnp.float32)]
            + [pltpu.VMEM((BLK,), jnp.int32) for _ in range(RING)]
            + [pltpu.VMEM((BLK, D), jnp.float32) for _ in range(RING)]
            + [pltpu.VMEM((D // 8, 8 * BLK), jnp.float32) for _ in range(RING)]
            + [pltpu.SemaphoreType.DMA for _ in range(3 * RING)]
        ),
    )(idx4, token_table, pos_table)


def kernel(inputs, token_table, pos_table):
    b, s = inputs.shape
    idx4 = (inputs.astype(jnp.int32)
            .reshape(b // 128, 128, s // 8, 8)
            .transpose(2, 0, 3, 1))
    out2 = _embed(idx4, token_table, pos_table)
    out5 = out2.reshape(s, D // 8, b // 128, 8, 128)
    return jnp.transpose(out5, (2, 4, 0, 1, 3)).reshape(b, s, D)


# speedup vs baseline: 1.0133x; 1.0133x over previous
import jax
import jax.numpy as jnp
from jax import lax
from jax.experimental import pallas as pl
from jax.experimental.pallas import tpu as pltpu
from jax.experimental.pallas import tpu_sc as plsc

SEQ = 200
D = 32
BATCH = 4096
SCALE = float(32.0 ** 0.5)

NC = 2
NS = 16
NW = NC * NS

BLK = 128
NBH = BATCH // BLK
N_TILES = SEQ * NBH
PER_W = N_TILES // NW
OUT_ROWS = SEQ * (D // 8) * NBH
RING = 4


def _lane_xor(x, k):
    idx = lax.iota(jnp.int32, 16) ^ k
    dn = lax.GatherDimensionNumbers(
        offset_dims=(), collapsed_slice_dims=(0,), start_index_map=(0,))
    return lax.gather(x, idx[:, None], dn, (1,),
                      mode=lax.GatherScatterMode.PROMISE_IN_BOUNDS)


def _transpose16(v):
    lane = lax.iota(jnp.int32, 16)
    for k in (8, 4, 2, 1):
        m = (lane & k) == 0
        nv = list(v)
        for i in range(16):
            if i & k:
                continue
            j = i | k
            x, y = v[i], v[j]
            mix = jnp.where(m, y, x)
            mixp = _lane_xor(mix, k)
            nv[i] = jnp.where(m, x, mixp)
            nv[j] = jnp.where(m, mixp, y)
        v = nv
    return v


def _body(idx_hbm, tok_hbm, pos_hbm, out_hbm, pos_v, *bufs):
    idx_v = bufs[0:RING]
    rows_v = bufs[RING:2 * RING]
    out_v = bufs[2 * RING:3 * RING]
    isem = bufs[3 * RING:4 * RING]
    gsem = bufs[4 * RING:5 * RING]
    osem = bufs[5 * RING:6 * RING]

    wid = lax.axis_index("s") * NC + lax.axis_index("c")
    base = wid * PER_W

    pltpu.sync_copy(pos_hbm, pos_v)

    def tile_sbh(g):
        t = base + g
        return t // NBH, t % NBH

    def idx_copy(g, r):
        s, bh = tile_sbh(g)
        return pltpu.make_async_copy(
            idx_hbm.at[s // 8, bh, s % 8], idx_v[r], isem[r])

    def gather_copy(r):
        return pltpu.make_async_copy(
            tok_hbm.at[idx_v[r]], rows_v[r], gsem[r])

    def out_copies(g, r):
        s, bh = tile_sbh(g)
        return [
            pltpu.make_async_copy(
                out_v[r].at[dh], out_hbm.at[(s * 4 + dh) * NBH + bh], osem[r])
            for dh in range(D // 8)
        ]

    def compute(g, r):
        s, _bh = tile_sbh(g)
        buf = rows_v[r]
        dst = out_v[r]

        plo = pos_v[s, pl.ds(0, 16)]
        phi = pos_v[s, pl.ds(16, 16)]

        def blk_body(bb, carry):
            b0 = bb * 16
            for db in range(2):
                pvec = plo if db == 0 else phi
                v = [buf[b0 + l, pl.ds(db * 16, 16)] * SCALE + pvec
                     for l in range(16)]
                t = _transpose16(v)
                for dd in range(16):
                    d = db * 16 + dd
                    dst[d // 8, pl.ds((d % 8) * BLK + b0, 16)] = t[dd]
            return carry

        lax.fori_loop(0, 8, blk_body, 0)

    for c in range(RING):
        idx_copy(c, c).start()
    for c in range(RING - 1):
        idx_copy(c, c).wait()
        gather_copy(c).start()

    def loop_body(go, carry):
        for rr in range(RING):
            g = go * RING + rr
            gather_copy(rr).wait()

            @pl.when(g + RING < PER_W)
            def _():
                idx_copy(g + RING, rr).start()

            @pl.when(g + RING - 1 < PER_W)
            def _():
                idx_copy(g + RING - 1, (rr + RING - 1) % RING).wait()
                gather_copy((rr + RING - 1) % RING).start()

            @pl.when(g >= RING)
            def _():
                for cp in out_copies(g - RING, rr):
                    cp.wait()

            compute(g, rr)
            for cp in out_copies(g, rr):
                cp.start()
        return carry

    lax.fori_loop(0, PER_W // RING, loop_body, 0)

    for rr in range(RING):
        for cp in out_copies(PER_W - RING + rr, rr):
            cp.wait()


@jax.jit
def _embed(idx4, token_table, pos_table):
    mesh = plsc.VectorSubcoreMesh(core_axis_name="c", subcore_axis_name="s")
    return pl.kernel(
        _body,
        out_type=jax.ShapeDtypeStruct((OUT_ROWS, 8 * BLK), jnp.float32),
        mesh=mesh,
        compiler_params=pltpu.CompilerParams(use_tc_tiling_on_sc=False),
        scratch_types=(
            [pltpu.VMEM((SEQ, D), jnp.float32)]
            + [pltpu.VMEM((BLK,), jnp.int32) for _ in range(RING)]
            + [pltpu.VMEM((BLK, D), jnp.float32) for _ in range(RING)]
            + [pltpu.VMEM((D // 8, 8 * BLK), jnp.float32) for _ in range(RING)]
            + [pltpu.SemaphoreType.DMA for _ in range(3 * RING)]
        ),
    )(idx4, token_table, pos_table)


def kernel(inputs, token_table, pos_table):
    b, s = inputs.shape
    idx4 = (inputs.astype(jnp.int32)
            .reshape(b // 128, 128, s // 8, 8)
            .transpose(2, 0, 3, 1))
    out2 = _embed(idx4, token_table, pos_table)
    out5 = out2.reshape(s, D // 8, b // 128, 8, 128)
    return jnp.transpose(out5, (2, 4, 0, 1, 3)).reshape(b, s, D)
